# baseline (speedup 1.0000x reference)
import jax
import jax.numpy as jnp
from jax.experimental import pallas as pl
from jax.experimental.pallas import tpu as pltpu

_K = 10


def _copy_body(x_hbm, b_ref, xo_hbm, bo_ref, vmem, in_sems, out_sems):
    n, d = x_hbm.shape
    rows = n // _K

    def _in(i):
        sl = pl.ds(i * rows, rows)
        return pltpu.make_async_copy(x_hbm.at[sl, :], vmem.at[sl, :], in_sems.at[i])

    def _out(i):
        sl = pl.ds(i * rows, rows)
        return pltpu.make_async_copy(vmem.at[sl, :], xo_hbm.at[sl, :], out_sems.at[i])

    for i in range(_K):
        _in(i).start()
    bo_ref[...] = b_ref[...]
    for i in range(_K):
        _in(i).wait()
        _out(i).start()
    for i in range(_K):
        _out(i).wait()


def kernel(x, edge_index, batch):
    del edge_index
    xo, bo = pl.pallas_call(
        _copy_body,
        in_specs=[
            pl.BlockSpec(memory_space=pltpu.MemorySpace.HBM),
            pl.BlockSpec(memory_space=pltpu.MemorySpace.VMEM),
        ],
        out_specs=(
            pl.BlockSpec(memory_space=pltpu.MemorySpace.HBM),
            pl.BlockSpec(memory_space=pltpu.MemorySpace.VMEM),
        ),
        out_shape=(
            jax.ShapeDtypeStruct(x.shape, x.dtype),
            jax.ShapeDtypeStruct(batch.shape, batch.dtype),
        ),
        scratch_shapes=[
            pltpu.VMEM(x.shape, x.dtype),
            pltpu.SemaphoreType.DMA((_K,)),
            pltpu.SemaphoreType.DMA((_K,)),
        ],
    )(x, batch)
    return (xo, bo)
